# baseline (speedup 1.0000x reference)
import jax
import jax.numpy as jnp
from jax.experimental import pallas as pl
from jax.experimental.pallas import tpu as pltpu

NUM_ADJS = 3
N = 4096
IN_F = 512
OUT_F = 512

CH = 256
NSTEPS = N // CH
NSLOT = 3


def _support_kernel(x_ref, w_ref, s_ref):
    prod = jnp.dot(
        x_ref[...].astype(jnp.bfloat16),
        w_ref[0].astype(jnp.bfloat16),
        preferred_element_type=jnp.float32,
    )
    s_ref[0] = (prod * (1.0 / NUM_ADJS)).astype(jnp.bfloat16)


def _chunk_copy(a_hbm, a_buf, sems, step, slot):
    return pltpu.make_async_copy(
        a_hbm.at[:, pl.ds(step * CH, CH), :],
        a_buf.at[slot],
        sems.at[slot],
    )


def _aggregate_kernel(a_hbm, s_ref, b_ref, o_ref, a_buf, sems):
    step = pl.program_id(0)

    @pl.when(step == 0)
    def _prologue():
        for j in range(NSLOT):
            _chunk_copy(a_hbm, a_buf, sems, j, j).start()

    slot = jax.lax.rem(step, NSLOT)
    _chunk_copy(a_hbm, a_buf, sems, step, slot).wait()

    acc = b_ref[...].astype(jnp.float32)
    for i in range(NUM_ADJS):
        acc = acc + jnp.dot(
            a_buf[slot, i].astype(jnp.bfloat16),
            s_ref[i],
            preferred_element_type=jnp.float32,
        )
    o_ref[...] = acc

    @pl.when(step + NSLOT < NSTEPS)
    def _refill():
        _chunk_copy(a_hbm, a_buf, sems, step + NSLOT, slot).start()


@jax.jit
def kernel(input_, adjs, adj_weight, bias):
    support = pl.pallas_call(
        _support_kernel,
        grid=(NUM_ADJS,),
        in_specs=[
            pl.BlockSpec((N, IN_F), lambda i: (0, 0)),
            pl.BlockSpec((1, IN_F, OUT_F), lambda i: (i, 0, 0)),
        ],
        out_specs=pl.BlockSpec((1, N, OUT_F), lambda i: (i, 0, 0)),
        out_shape=jax.ShapeDtypeStruct((NUM_ADJS, N, OUT_F), jnp.bfloat16),
    )(input_, adj_weight)

    bias2d = bias.reshape(1, OUT_F)

    out = pl.pallas_call(
        _aggregate_kernel,
        grid=(NSTEPS,),
        in_specs=[
            pl.BlockSpec(memory_space=pl.ANY),
            pl.BlockSpec((NUM_ADJS, N, OUT_F), lambda m: (0, 0, 0)),
            pl.BlockSpec((1, OUT_F), lambda m: (0, 0)),
        ],
        out_specs=pl.BlockSpec((CH, OUT_F), lambda m: (m, 0)),
        out_shape=jax.ShapeDtypeStruct((N, OUT_F), jnp.float32),
        scratch_shapes=[
            pltpu.VMEM((NSLOT, NUM_ADJS, CH, N), jnp.float32),
            pltpu.SemaphoreType.DMA((NSLOT,)),
        ],
    )(adjs, support, bias2d)

    return out
